# baseline (speedup 1.0000x reference)
import functools

import jax
import jax.numpy as jnp
from jax import lax
from jax.experimental import pallas as pl
from jax.experimental.pallas import tpu as pltpu
from jax.experimental.pallas import tpu_sc as plsc

N_NODES = 10000
N_EDGES = 320000
F_IN = 128
HID = 10
FP = 16

NW = 32
NS = 16
CHUNK = 128
CPT = 79
EPT = CPT * CHUNK
EPAD = NW * EPT
NPAD = 10240
NROWS = NPAD // 16
STRIPE = NROWS // NS
HPT = EPAD // NS
HGPT = HPT // 16
NBUF = 8


@functools.cache
def _sc_mesh():
    return plsc.VectorSubcoreMesh(
        core_axis_name="c", subcore_axis_name="s", num_cores=2, num_subcores=16)


def _sc_body(row_hbm, col_hbm, y1_hbm, zb_hbm, idx_hbm,
             acc_out, z_out, dis_out,
             rowb_v, deg_v, row_v, col_v, y1s_v, idx_v,
             *rest):
    bufs = rest[:NBUF]
    deg2d_s = rest[NBUF]
    acc_s = rest[NBUF + 1]
    gsems = rest[NBUF + 2:2 * NBUF + 2]
    ssems = rest[2 * NBUF + 2:]

    cid = lax.axis_index("c")
    sid = lax.axis_index("s")
    wid = sid * 2 + cid

    pltpu.sync_copy(row_hbm.at[wid], row_v)
    pltpu.sync_copy(row_hbm.at[sid * 2 + (1 - cid)], rowb_v)
    zeros16 = jnp.zeros((16,), jnp.float32)
    ones16 = jnp.ones((16,), jnp.float32)

    def zero_body(j, carry):
        deg_v[j] = zeros16
        return carry

    lax.fori_loop(0, NROWS, zero_body, 0, unroll=8)

    def hist_body(j, carry):
        for buf in (row_v, rowb_v):
            for l in range(CHUNK // 16):
                idx = buf[j, pl.ds(l * 16, 16)]
                plsc.addupdate_scatter(deg_v, [idx >> 4, idx & 15], ones16)
        return carry

    lax.fori_loop(0, CPT, hist_body, 0)

    pltpu.sync_copy(idx_hbm, idx_v)
    pltpu.sync_copy(zb_hbm.at[pl.ds(0, STRIPE)],
                    deg2d_s.at[pl.ds(sid * STRIPE, STRIPE)])
    plsc.subcore_barrier()
    for c in range(NROWS // CHUNK):
        pltpu.sync_copy(deg_v.at[pl.ds(c * CHUNK, CHUNK)],
                        deg2d_s.at[idx_v.at[c]], add=True)
    plsc.subcore_barrier()

    pltpu.sync_copy(deg2d_s.at[pl.ds(sid * STRIPE, STRIPE)],
                    deg_v.at[pl.ds(0, STRIPE)])

    def newton_body(r, carry):
        d = deg_v[r]
        i = plsc.bitcast(d, jnp.int32)
        y = plsc.bitcast(jnp.int32(0x5F3759DF) - (i >> 1), jnp.float32)
        half = d * -0.5
        for _ in range(3):
            y = y * (half * y * y + 1.5)
        deg_v[r] = jnp.where(d > 0.0, y, 0.0)
        return carry

    lax.fori_loop(0, STRIPE, newton_body, 0, unroll=4)
    pltpu.sync_copy(deg_v.at[pl.ds(0, STRIPE)],
                    dis_out.at[pl.ds(sid * STRIPE, STRIPE)])

    pltpu.sync_copy(y1_hbm.at[pl.ds(sid * STRIPE * 16, STRIPE * 16)], y1s_v)

    zeros16i = jnp.zeros((16,), jnp.int32)

    def scale_body(r, carry):
        for l in range(16):
            d = plsc.load_gather(deg_v, [zeros16i + r, zeros16i + l])
            k = r * 16 + l
            y1s_v[k] = y1s_v[k] * d
        return carry

    lax.fori_loop(0, STRIPE, scale_body, 0)
    pltpu.sync_copy(y1s_v, z_out.at[pl.ds(sid * STRIPE * 16, STRIPE * 16)])

    pltpu.sync_copy(col_hbm.at[wid], col_v)
    pltpu.sync_copy(zb_hbm, acc_s.at[pl.ds(sid * NROWS, NROWS)])
    plsc.subcore_barrier()

    for i in range(NBUF):
        pltpu.async_copy(z_out.at[row_v.at[i]], bufs[i], gsems[i])

    def body(k, carry):
        j0 = NBUF * k
        for i in range(NBUF):
            pltpu.make_async_copy(
                z_out.at[row_v.at[j0 + i]], bufs[i], gsems[i]).wait()
            pltpu.async_copy(
                bufs[i], acc_s.at[col_v.at[j0 + i]], ssems[i], add=True)
        for i in range(NBUF):
            pltpu.make_async_copy(
                bufs[i], acc_s.at[col_v.at[j0 + i]], ssems[i]).wait()
            pltpu.async_copy(
                z_out.at[row_v.at[j0 + NBUF + i]], bufs[i], gsems[i])
        return carry

    lax.fori_loop(0, CPT // NBUF - 1, body, 0)
    j0 = (CPT // NBUF - 1) * NBUF
    for i in range(NBUF):
        pltpu.make_async_copy(
            z_out.at[row_v.at[j0 + i]], bufs[i], gsems[i]).wait()
        pltpu.async_copy(
            bufs[i], acc_s.at[col_v.at[j0 + i]], ssems[i], add=True)
    for i in range(NBUF):
        pltpu.make_async_copy(
            bufs[i], acc_s.at[col_v.at[j0 + i]], ssems[i]).wait()
    for j in range(NBUF * (CPT // NBUF), CPT):
        i = j % NBUF
        pltpu.sync_copy(z_out.at[row_v.at[j]], bufs[i])
        pltpu.sync_copy(bufs[i], acc_s.at[col_v.at[j]], add=True)
    plsc.subcore_barrier()

    pltpu.sync_copy(acc_s.at[pl.ds(sid * NROWS, NROWS)],
                    acc_out.at[cid, pl.ds(sid * NROWS, NROWS)])


@functools.cache
def _sc_kernel():
    return pl.kernel(
        _sc_body,
        out_type=(
            jax.ShapeDtypeStruct((2, NPAD, FP), jnp.float32),
            jax.ShapeDtypeStruct((NPAD, FP), jnp.float32),
            jax.ShapeDtypeStruct((NROWS, 16), jnp.float32),
        ),
        mesh=_sc_mesh(),
        scratch_types=[
            pltpu.VMEM((CPT, CHUNK), jnp.int32),
            pltpu.VMEM((NROWS, 16), jnp.float32),
            pltpu.VMEM((CPT, CHUNK), jnp.int32),
            pltpu.VMEM((CPT, CHUNK), jnp.int32),
            pltpu.VMEM((STRIPE * 16, FP), jnp.float32),
            pltpu.VMEM((NROWS // CHUNK, CHUNK), jnp.int32),
        ] + [pltpu.VMEM((CHUNK, FP), jnp.float32)] * NBUF + [
            pltpu.VMEM_SHARED((NROWS, 16), jnp.float32),
            pltpu.VMEM_SHARED((NPAD, FP), jnp.float32),
        ] + [pltpu.SemaphoreType.DMA] * (2 * NBUF),
        compiler_params=pltpu.CompilerParams(
            needs_layout_passes=False, use_tc_tiling_on_sc=False),
    )


def _tc_a_body(xp_ref, w0_ref, w1_ref, y0_ref, y1_ref):
    x = xp_ref[...]
    y0_ref[...] = jnp.dot(x, w0_ref[...], preferred_element_type=jnp.float32)
    y1_ref[...] = jnp.dot(x, w1_ref[...], preferred_element_type=jnp.float32)


def _tc_a(xp, w0p, w1p):
    return pl.pallas_call(
        _tc_a_body,
        out_shape=(
            jax.ShapeDtypeStruct((NPAD, FP), jnp.float32),
            jax.ShapeDtypeStruct((NPAD, FP), jnp.float32),
        ),
    )(xp, w0p, w1p)


def _tc_b_body(acc_ref, y0_ref, dis_ref, bp_ref, wfc_ref, bfc_ref, out_ref):
    accsum = acc_ref[0] + acc_ref[1]
    dis = dis_ref[...]
    pre = y0_ref[...] - accsum * dis[:, None] + bp_ref[...]
    h = jnp.maximum(pre, 0.0)
    logits = jnp.dot(h, wfc_ref[...], preferred_element_type=jnp.float32)
    logits = logits + bfc_ref[...]
    lane = lax.broadcasted_iota(jnp.int32, logits.shape, 1)
    masked = jnp.where(lane < HID, logits, -jnp.inf)
    m = jnp.max(masked, axis=1, keepdims=True)
    s = jnp.sum(jnp.exp(masked - m), axis=1, keepdims=True)
    out_ref[...] = logits - m - jnp.log(s)


def _tc_b(acc, y0, dis, bp, wfcp, bfcp):
    return pl.pallas_call(
        _tc_b_body,
        out_shape=jax.ShapeDtypeStruct((NPAD, FP), jnp.float32),
    )(acc, y0, dis, bp, wfcp, bfcp)


@jax.jit
def kernel(x, edge_index, W0, W1, b, Wfc, bfc):
    row = edge_index[0]
    col = edge_index[1]
    pad = jnp.full((EPAD - N_EDGES,), N_NODES, jnp.int32)
    rowp = jnp.concatenate([row, pad])
    colp = jnp.concatenate([col, pad])
    row3 = rowp.reshape(NW, CPT, CHUNK)
    col3 = colp.reshape(NW, CPT, CHUNK)
    xp = jnp.pad(x, ((0, NPAD - N_NODES), (0, 0)))
    w0p = jnp.pad(W0, ((0, 0), (0, FP - HID)))
    w1p = jnp.pad(W1, ((0, 0), (0, FP - HID)))
    bp = jnp.pad(b, (0, FP - HID)).reshape(1, FP)
    wfcp = jnp.pad(Wfc, ((0, FP - HID), (0, FP - HID)))
    bfcp = jnp.pad(bfc, (0, FP - HID)).reshape(1, FP)
    zb = jnp.zeros((NROWS, FP), jnp.float32)
    idx5 = jnp.arange(NROWS, dtype=jnp.int32).reshape(NROWS // CHUNK, CHUNK)

    y0, y1 = _tc_a(xp, w0p, w1p)
    acc, _z, dis = _sc_kernel()(row3, col3, y1, zb, idx5)
    res = _tc_b(acc, y0, dis.reshape(NPAD), bp, wfcp, bfcp)
    return res[:N_NODES, :HID]
